# baseline (speedup 1.0000x reference)
import functools

import jax
import jax.numpy as jnp
from jax import lax
from jax.experimental import pallas as pl
from jax.experimental.pallas import tpu as pltpu
from jax.experimental.pallas import tpu_sc as plsc

NLABELS = 1000000
EMBED_DIM = 64
BATCH = 16384

_INFO = plsc.get_sparse_core_info()
_NC = _INFO.num_cores
_NS = _INFO.num_subcores
_L = _INFO.num_lanes
_NW = _NC * _NS
_NTC = (NLABELS + 127) // 128
_TCPW = (_NTC + _NW - 1) // _NW
_CAP = 2048
_BPAD = 129
_RING = 4
_NBK = _TCPW + 16
_OPAD = BATCH + 2 * _L


def _splat(x, dtype=jnp.int32):
    return jnp.full((_L,), x, dtype=dtype)


def _rsqrt16(x):
    i = plsc.bitcast(x, jnp.int32)
    i = jnp.int32(0x5F3759DF) - lax.shift_right_arithmetic(i, jnp.int32(1))
    y = plsc.bitcast(i, jnp.float32)
    for _ in range(2):
        y = y * (jnp.float32(1.5) - jnp.float32(0.5) * x * y * y)
    return y


def _sc_body(y_hbm, tT_hbm, out_hbm, yva, ylist, blist, cnt, boffa, boffb,
             perm, bperm, dlist, dlo, dhi, blk, ostage,
             sema, semb, semc, semd, semsa, semsb):
    wid = lax.axis_index("s") * _NC + lax.axis_index("c")
    tclo = wid * _TCPW
    lane = lax.iota(jnp.int32, _L)
    zeros16 = jnp.zeros((_L,), jnp.int32)
    tclov = _splat(tclo)
    tchiv = tclov + _TCPW

    pltpu.sync_copy(y_hbm.at[pl.ds(0, BATCH)], yva)

    def precount(i, runv):
        yv = yva[pl.ds(i * _L, _L)]
        tcv = lax.shift_right_logical(yv, 7)
        mk = (tcv >= tclov) & (tcv < tchiv)
        return runv + plsc.all_reduce_population_count(mk)

    mtot = lax.fori_loop(0, BATCH // _L, precount, zeros16)[0]
    nrounds = lax.shift_right_logical(mtot + jnp.int32(_CAP - 1), 11)

    sems = [sema, semb, semc, semd]
    ssems = [semsa, semsb]
    trash = _splat(BATCH) + lane

    def issue(s, pb, sem):
        tcs = dlist[pl.ds(s, _L)][0]
        off = pl.multiple_of(tcs * 128, 128)
        pltpu.async_copy(tT_hbm.at[:, pl.ds(off, 128)],
                         blk.at[pb, :, pl.ds(0, 128)], sem)

    def wait(s, pb, sem):
        tcs = dlist[pl.ds(s, _L)][0]
        off = pl.multiple_of(tcs * 128, 128)
        pltpu.make_async_copy(tT_hbm.at[:, pl.ds(off, 128)],
                              blk.at[pb, :, pl.ds(0, 128)], sem).wait()

    def sc_issue(chunk, sl):
        pltpu.async_copy(ostage.at[sl],
                         out_hbm.at[bperm.at[chunk]], ssems[sl])

    def sc_wait(sl):
        pltpu.make_async_copy(ostage.at[sl],
                              out_hbm.at[bperm.at[0]], ssems[sl]).wait()

    def pass_body(rnd, c0):
        rlo = rnd * _CAP
        rlov = _splat(rlo)
        rhiv = rlov + _CAP

        def zero_cnt(i, c):
            cnt[pl.ds(i * _L, _L)] = zeros16
            return c

        lax.fori_loop(0, _NBK // _L, zero_cnt, jnp.int32(0))

        def fill_bperm(i, c):
            bperm[i % 16, pl.ds((i // 16) * _L, _L)] = trash
            return c

        lax.fori_loop(0, _CAP // _L, fill_bperm, jnp.int32(0))

        def scan_body(i, runv):
            yv = yva[pl.ds(i * _L, _L)]
            tcv = lax.shift_right_logical(yv, 7)
            mk = (tcv >= tclov) & (tcv < tchiv)
            mki = jnp.where(mk, jnp.int32(1), jnp.int32(0))
            gpos = runv + plsc.cumsum(mki) - jnp.int32(1)
            wmk = mk & (gpos >= rlov) & (gpos < rhiv)
            loff = jnp.clip(runv[0] - rlo, 0, _CAP)
            plsc.store_compressed(ylist.at[pl.ds(loff, _L)], yv, mask=wmk)
            plsc.store_compressed(blist.at[pl.ds(loff, _L)], i * _L + lane,
                                  mask=wmk)
            return runv + plsc.all_reduce_population_count(mk)

        lax.fori_loop(0, BATCH // _L, scan_body, zeros16)
        m = jnp.minimum(jnp.int32(_CAP), mtot - rlo)

        nev = lax.shift_right_logical(m + jnp.int32(_L - 1), 4)
        mv = _splat(m)

        def count_body(v, c):
            valid = (v * _L + lane) < mv
            yvv = ylist[pl.ds(v * _L, _L)]
            bk = lax.shift_right_logical(yvv, 7) - tclov
            cnts, lastm = plsc.scan_count(bk, mask=valid)
            w = plsc.load_gather(cnt, [bk], mask=lastm)
            plsc.store_scatter(cnt, [bk], w + cnts, mask=lastm)
            return c

        lax.fori_loop(0, nev, count_body, jnp.int32(0))

        def boff_body(i, runv):
            v = cnt[pl.ds(i * _L, _L)]
            cs = plsc.cumsum(v)
            ex = runv + cs - v
            boffa[pl.ds(i * _L, _L)] = ex
            boffb[pl.ds(i * _L, _L)] = ex
            return _splat((runv + cs)[15])

        lax.fori_loop(0, _NBK // _L, boff_body, zeros16)

        def compact_body(i, runv):
            cv = cnt[pl.ds(i * _L, _L)]
            mk = cv > 0
            mki = jnp.where(mk, jnp.int32(1), jnp.int32(0))
            cs = plsc.cumsum(mki)
            slots = runv + cs - mki
            tcg = tclov + i * _L + lane
            bo = boffa[pl.ds(i * _L, _L)]
            plsc.store_scatter(dlist, [slots], tcg, mask=mk)
            plsc.store_scatter(dlo, [slots], bo, mask=mk)
            plsc.store_scatter(dhi, [slots], bo + cv, mask=mk)
            return _splat((runv + cs)[15])

        nblocks = lax.fori_loop(0, _NBK // _L, compact_body, zeros16)[0]

        def place_body(v, c):
            valid = (v * _L + lane) < mv
            yvv = ylist[pl.ds(v * _L, _L)]
            bk = lax.shift_right_logical(yvv, 7) - tclov
            cnts, lastm = plsc.scan_count(bk, mask=valid)
            base = plsc.load_gather(boffb, [bk], mask=valid)
            pos = base + cnts - jnp.int32(1)
            plsc.store_scatter(boffb, [bk], base + cnts, mask=lastm)
            plsc.store_scatter(perm, [pos], yvv, mask=valid)
            bvv = blist[pl.ds(v * _L, _L)]
            plsc.store_scatter(
                bperm,
                [lax.shift_right_logical(pos, 7), pos & jnp.int32(127)],
                bvv, mask=valid)
            return c

        lax.fori_loop(0, nev, place_body, jnp.int32(0))

        def process(s, pb):
            lo_e = dlo[pl.ds(s, _L)][0]
            hi_e = dhi[pl.ds(s, _L)][0]
            pbv = _splat(pb)

            def ebody(pos, c):
                r = pos & jnp.int32(127)
                chunk = lax.shift_right_logical(pos, 7)
                sl = chunk & jnp.int32(1)

                @pl.when((r == 0) & (pos >= 256))
                def _():
                    @pl.when(sl == 0)
                    def _():
                        sc_wait(0)

                    @pl.when(sl == 1)
                    def _():
                        sc_wait(1)

                ye = perm[pl.ds(pos, _L)][0]
                colv = _splat(ye & jnp.int32(127))
                gs = []
                acc = jnp.zeros((_L,), jnp.float32)
                for g in range(4):
                    gv = plsc.load_gather(blk, [pbv, lane + _L * g, colv])
                    gs.append(gv)
                    acc = acc + gv * gv
                scale = _rsqrt16(_splat(jnp.sum(acc), jnp.float32))
                for g in range(4):
                    ostage[sl, r, pl.ds(_L * g, _L)] = gs[g] * scale

                @pl.when(r == 127)
                def _():
                    @pl.when(sl == 0)
                    def _():
                        sc_issue(chunk, 0)

                    @pl.when(sl == 1)
                    def _():
                        sc_issue(chunk, 1)

                return c

            lax.fori_loop(lo_e, hi_e, ebody, jnp.int32(0))

        for j in range(_RING):
            @pl.when(jnp.int32(j) < nblocks)
            def _(j=j):
                issue(jnp.int32(j), j, sems[j])

        nring = lax.div(nblocks + jnp.int32(_RING - 1), jnp.int32(_RING))

        def ring_body(q, c):
            s0 = q * _RING
            for j in range(_RING):
                sj = s0 + j

                @pl.when(sj < nblocks)
                def _(j=j, sj=sj):
                    wait(sj, j, sems[j])
                    process(sj, j)

                    @pl.when(sj + _RING < nblocks)
                    def _(j=j, sj=sj):
                        issue(sj + _RING, j, sems[j])

            return c

        lax.fori_loop(0, nring, ring_body, jnp.int32(0))

        lastc = lax.shift_right_logical(m, 7)
        lsl = lastc & jnp.int32(1)

        @pl.when((m & jnp.int32(127)) != 0)
        def _():
            @pl.when(lsl == 0)
            def _():
                sc_issue(lastc, 0)

            @pl.when(lsl == 1)
            def _():
                sc_issue(lastc, 1)

        nchunks = lax.shift_right_logical(m + jnp.int32(127), 7)
        for p in range(2):
            @pl.when((nchunks >= 1) & (((nchunks - 1) & jnp.int32(1)) == p))
            def _(p=p):
                sc_wait(p)

            @pl.when((nchunks >= 2) & (((nchunks - 2) & jnp.int32(1)) == p))
            def _(p=p):
                sc_wait(p)

        return c0

    lax.fori_loop(0, nrounds, pass_body, jnp.int32(0))


@jax.jit
def kernel(y, table):
    mesh = plsc.VectorSubcoreMesh(core_axis_name="c", subcore_axis_name="s")
    k = functools.partial(
        pl.kernel,
        mesh=mesh,
        compiler_params=pltpu.CompilerParams(needs_layout_passes=False),
        out_type=jax.ShapeDtypeStruct((_OPAD, 2 * EMBED_DIM), jnp.float32),
        scratch_types=[
            pltpu.VMEM((BATCH,), jnp.int32),
            pltpu.VMEM((_CAP + _L,), jnp.int32),
            pltpu.VMEM((_CAP + _L,), jnp.int32),
            pltpu.VMEM((_NBK,), jnp.int32),
            pltpu.VMEM((_NBK,), jnp.int32),
            pltpu.VMEM((_NBK,), jnp.int32),
            pltpu.VMEM((_CAP + _L,), jnp.int32),
            pltpu.VMEM((_CAP // 128, 128), jnp.int32),
            pltpu.VMEM((_NBK,), jnp.int32),
            pltpu.VMEM((_NBK,), jnp.int32),
            pltpu.VMEM((_NBK,), jnp.int32),
            pltpu.VMEM((_RING, EMBED_DIM, _BPAD), jnp.float32),
            pltpu.VMEM((2, 128, 2 * EMBED_DIM), jnp.float32),
            pltpu.SemaphoreType.DMA,
            pltpu.SemaphoreType.DMA,
            pltpu.SemaphoreType.DMA,
            pltpu.SemaphoreType.DMA,
            pltpu.SemaphoreType.DMA,
            pltpu.SemaphoreType.DMA,
        ],
    )(_sc_body)
    out2 = k(y.astype(jnp.int32), table.T)
    return out2[:BATCH, :EMBED_DIM]


# speedup vs baseline: 1.0171x; 1.0171x over previous
import functools

import jax
import jax.numpy as jnp
from jax import lax
from jax.experimental import pallas as pl
from jax.experimental.pallas import tpu as pltpu
from jax.experimental.pallas import tpu_sc as plsc

NLABELS = 1000000
EMBED_DIM = 64
BATCH = 16384

_INFO = plsc.get_sparse_core_info()
_NC = _INFO.num_cores
_NS = _INFO.num_subcores
_L = _INFO.num_lanes
_NW = _NC * _NS
_NTC = (NLABELS + 127) // 128
_TCPW = (_NTC + _NW - 1) // _NW
_CAP = 2048
_BPAD = 129
_RING = 4
_NBK = _TCPW + 16
_OPAD = BATCH + 2 * _L


def _splat(x, dtype=jnp.int32):
    return jnp.full((_L,), x, dtype=dtype)


def _rsqrt16(x):
    i = plsc.bitcast(x, jnp.int32)
    i = jnp.int32(0x5F3759DF) - lax.shift_right_arithmetic(i, jnp.int32(1))
    y = plsc.bitcast(i, jnp.float32)
    for _ in range(2):
        y = y * (jnp.float32(1.5) - jnp.float32(0.5) * x * y * y)
    return y


def _sc_body(y_hbm, tT_hbm, out_hbm, yva, ylist, blist, cnt, boffa, boffb,
             perm, bperm, dlist, dlo, dhi, blk, ostage,
             sema, semb, semc, semd, semsa, semsb):
    wid = lax.axis_index("s") * _NC + lax.axis_index("c")
    tclo = wid * _TCPW
    lane = lax.iota(jnp.int32, _L)
    zeros16 = jnp.zeros((_L,), jnp.int32)
    tclov = _splat(tclo)
    tchiv = tclov + _TCPW

    pltpu.sync_copy(y_hbm.at[pl.ds(0, BATCH)], yva)

    sems = [sema, semb, semc, semd]
    ssems = [semsa, semsb]
    trash = _splat(BATCH) + lane

    def issue(s, pb, sem):
        tcs = dlist[pl.ds(s, _L)][0]
        off = pl.multiple_of(tcs * 128, 128)
        pltpu.async_copy(tT_hbm.at[:, pl.ds(off, 128)],
                         blk.at[pb, :, pl.ds(0, 128)], sem)

    def wait(s, pb, sem):
        tcs = dlist[pl.ds(s, _L)][0]
        off = pl.multiple_of(tcs * 128, 128)
        pltpu.make_async_copy(tT_hbm.at[:, pl.ds(off, 128)],
                              blk.at[pb, :, pl.ds(0, 128)], sem).wait()

    def sc_issue(chunk, sl):
        pltpu.async_copy(ostage.at[sl],
                         out_hbm.at[bperm.at[chunk]], ssems[sl])

    def sc_wait(sl):
        pltpu.make_async_copy(ostage.at[sl],
                              out_hbm.at[bperm.at[0]], ssems[sl]).wait()

    def do_round(rnd, mtot_in):
        rlo = rnd * _CAP
        rlov = _splat(rlo)
        rhiv = rlov + _CAP

        def zero_cnt(i, c):
            cnt[pl.ds(i * _L, _L)] = zeros16
            return c

        lax.fori_loop(0, _NBK // _L, zero_cnt, jnp.int32(0))

        def fill_bperm(i, c):
            bperm[i % 16, pl.ds((i // 16) * _L, _L)] = trash
            return c

        lax.fori_loop(0, _CAP // _L, fill_bperm, jnp.int32(0))

        def scan_body(i, runv):
            yv = yva[pl.ds(i * _L, _L)]
            tcv = lax.shift_right_logical(yv, 7)
            mk = (tcv >= tclov) & (tcv < tchiv)
            mki = jnp.where(mk, jnp.int32(1), jnp.int32(0))
            gpos = runv + plsc.cumsum(mki) - jnp.int32(1)
            wmk = mk & (gpos >= rlov) & (gpos < rhiv)
            loff = jnp.clip(runv[0] - rlo, 0, _CAP)
            plsc.store_compressed(ylist.at[pl.ds(loff, _L)], yv, mask=wmk)
            plsc.store_compressed(blist.at[pl.ds(loff, _L)], i * _L + lane,
                                  mask=wmk)
            return runv + plsc.all_reduce_population_count(mk)

        mtot = lax.fori_loop(0, BATCH // _L, scan_body, zeros16)[0]
        m = jnp.minimum(jnp.int32(_CAP),
                        jnp.maximum(mtot, mtot_in) - rlo)

        nev = lax.shift_right_logical(m + jnp.int32(_L - 1), 4)
        mv = _splat(m)

        def count_body(v, c):
            valid = (v * _L + lane) < mv
            yvv = ylist[pl.ds(v * _L, _L)]
            bk = lax.shift_right_logical(yvv, 7) - tclov
            cnts, lastm = plsc.scan_count(bk, mask=valid)
            w = plsc.load_gather(cnt, [bk], mask=lastm)
            plsc.store_scatter(cnt, [bk], w + cnts, mask=lastm)
            return c

        lax.fori_loop(0, nev, count_body, jnp.int32(0))

        def boff_body(i, runv):
            v = cnt[pl.ds(i * _L, _L)]
            cs = plsc.cumsum(v)
            ex = runv + cs - v
            boffa[pl.ds(i * _L, _L)] = ex
            boffb[pl.ds(i * _L, _L)] = ex
            return _splat((runv + cs)[15])

        lax.fori_loop(0, _NBK // _L, boff_body, zeros16)

        def compact_body(i, runv):
            cv = cnt[pl.ds(i * _L, _L)]
            mk = cv > 0
            mki = jnp.where(mk, jnp.int32(1), jnp.int32(0))
            cs = plsc.cumsum(mki)
            slots = runv + cs - mki
            tcg = tclov + i * _L + lane
            bo = boffa[pl.ds(i * _L, _L)]
            plsc.store_scatter(dlist, [slots], tcg, mask=mk)
            plsc.store_scatter(dlo, [slots], bo, mask=mk)
            plsc.store_scatter(dhi, [slots], bo + cv, mask=mk)
            return _splat((runv + cs)[15])

        nblocks = lax.fori_loop(0, _NBK // _L, compact_body, zeros16)[0]

        def place_body(v, c):
            valid = (v * _L + lane) < mv
            yvv = ylist[pl.ds(v * _L, _L)]
            bk = lax.shift_right_logical(yvv, 7) - tclov
            cnts, lastm = plsc.scan_count(bk, mask=valid)
            base = plsc.load_gather(boffb, [bk], mask=valid)
            pos = base + cnts - jnp.int32(1)
            plsc.store_scatter(boffb, [bk], base + cnts, mask=lastm)
            plsc.store_scatter(perm, [pos], yvv, mask=valid)
            bvv = blist[pl.ds(v * _L, _L)]
            plsc.store_scatter(
                bperm,
                [lax.shift_right_logical(pos, 7), pos & jnp.int32(127)],
                bvv, mask=valid)
            return c

        lax.fori_loop(0, nev, place_body, jnp.int32(0))

        def process(s, pb):
            lo_e = dlo[pl.ds(s, _L)][0]
            hi_e = dhi[pl.ds(s, _L)][0]
            pbv = _splat(pb)

            def ebody(pos, c):
                r = pos & jnp.int32(127)
                chunk = lax.shift_right_logical(pos, 7)
                sl = chunk & jnp.int32(1)

                @pl.when((r == 0) & (pos >= 256))
                def _():
                    @pl.when(sl == 0)
                    def _():
                        sc_wait(0)

                    @pl.when(sl == 1)
                    def _():
                        sc_wait(1)

                ye = perm[pl.ds(pos, _L)][0]
                colv = _splat(ye & jnp.int32(127))
                gs = []
                acc = jnp.zeros((_L,), jnp.float32)
                for g in range(4):
                    gv = plsc.load_gather(blk, [pbv, lane + _L * g, colv])
                    gs.append(gv)
                    acc = acc + gv * gv
                scale = _rsqrt16(_splat(jnp.sum(acc), jnp.float32))
                for g in range(4):
                    ostage[sl, r, pl.ds(_L * g, _L)] = gs[g] * scale

                @pl.when(r == 127)
                def _():
                    @pl.when(sl == 0)
                    def _():
                        sc_issue(chunk, 0)

                    @pl.when(sl == 1)
                    def _():
                        sc_issue(chunk, 1)

                return c

            lax.fori_loop(lo_e, hi_e, ebody, jnp.int32(0))

        for j in range(_RING):
            @pl.when(jnp.int32(j) < nblocks)
            def _(j=j):
                issue(jnp.int32(j), j, sems[j])

        nring = lax.div(nblocks + jnp.int32(_RING - 1), jnp.int32(_RING))

        def ring_body(q, c):
            s0 = q * _RING
            for j in range(_RING):
                sj = s0 + j

                @pl.when(sj < nblocks)
                def _(j=j, sj=sj):
                    wait(sj, j, sems[j])
                    process(sj, j)

                    @pl.when(sj + _RING < nblocks)
                    def _(j=j, sj=sj):
                        issue(sj + _RING, j, sems[j])

            return c

        lax.fori_loop(0, nring, ring_body, jnp.int32(0))

        lastc = lax.shift_right_logical(m, 7)
        lsl = lastc & jnp.int32(1)

        @pl.when((m & jnp.int32(127)) != 0)
        def _():
            @pl.when(lsl == 0)
            def _():
                sc_issue(lastc, 0)

            @pl.when(lsl == 1)
            def _():
                sc_issue(lastc, 1)

        nchunks = lax.shift_right_logical(m + jnp.int32(127), 7)
        for p in range(2):
            @pl.when((nchunks >= 1) & (((nchunks - 1) & jnp.int32(1)) == p))
            def _(p=p):
                sc_wait(p)

            @pl.when((nchunks >= 2) & (((nchunks - 2) & jnp.int32(1)) == p))
            def _(p=p):
                sc_wait(p)

        return mtot

    mtot0 = do_round(jnp.int32(0), jnp.int32(0))

    def more_cond(st):
        rnd, mtot = st
        return (rnd + 1) * _CAP < mtot

    def more_body(st):
        rnd, mtot = st
        rnd = rnd + 1
        do_round(rnd, mtot)
        return (rnd, mtot)

    lax.while_loop(more_cond, more_body, (jnp.int32(0), mtot0))


@jax.jit
def kernel(y, table):
    mesh = plsc.VectorSubcoreMesh(core_axis_name="c", subcore_axis_name="s")
    k = functools.partial(
        pl.kernel,
        mesh=mesh,
        compiler_params=pltpu.CompilerParams(needs_layout_passes=False),
        out_type=jax.ShapeDtypeStruct((_OPAD, 2 * EMBED_DIM), jnp.float32),
        scratch_types=[
            pltpu.VMEM((BATCH,), jnp.int32),
            pltpu.VMEM((_CAP + _L,), jnp.int32),
            pltpu.VMEM((_CAP + _L,), jnp.int32),
            pltpu.VMEM((_NBK,), jnp.int32),
            pltpu.VMEM((_NBK,), jnp.int32),
            pltpu.VMEM((_NBK,), jnp.int32),
            pltpu.VMEM((_CAP + _L,), jnp.int32),
            pltpu.VMEM((_CAP // 128, 128), jnp.int32),
            pltpu.VMEM((_NBK,), jnp.int32),
            pltpu.VMEM((_NBK,), jnp.int32),
            pltpu.VMEM((_NBK,), jnp.int32),
            pltpu.VMEM((_RING, EMBED_DIM, _BPAD), jnp.float32),
            pltpu.VMEM((2, 128, 2 * EMBED_DIM), jnp.float32),
            pltpu.SemaphoreType.DMA,
            pltpu.SemaphoreType.DMA,
            pltpu.SemaphoreType.DMA,
            pltpu.SemaphoreType.DMA,
            pltpu.SemaphoreType.DMA,
            pltpu.SemaphoreType.DMA,
        ],
    )(_sc_body)
    out2 = k(y.astype(jnp.int32), table.T)
    return out2[:BATCH, :EMBED_DIM]


# speedup vs baseline: 1.0634x; 1.0455x over previous
import functools

import jax
import jax.numpy as jnp
from jax import lax
from jax.experimental import pallas as pl
from jax.experimental.pallas import tpu as pltpu
from jax.experimental.pallas import tpu_sc as plsc

NLABELS = 1000000
EMBED_DIM = 64
BATCH = 16384

_INFO = plsc.get_sparse_core_info()
_NC = _INFO.num_cores
_NS = _INFO.num_subcores
_L = _INFO.num_lanes
_NW = _NC * _NS
_NTC = (NLABELS + 127) // 128
_TCPW = (_NTC + _NW - 1) // _NW
_CAP = 2048
_BPAD = 129
_RING = 4
_NBK = _TCPW + 16
_OPAD = BATCH + 2 * _L


def _splat(x, dtype=jnp.int32):
    return jnp.full((_L,), x, dtype=dtype)


def _rsqrt16(x):
    i = plsc.bitcast(x, jnp.int32)
    i = jnp.int32(0x5F3759DF) - lax.shift_right_arithmetic(i, jnp.int32(1))
    y = plsc.bitcast(i, jnp.float32)
    for _ in range(2):
        y = y * (jnp.float32(1.5) - jnp.float32(0.5) * x * y * y)
    return y


def _sc_body(y_hbm, tT_hbm, out_hbm, yva, ylist, blist, cnt, boffa, boffb,
             perm, bperm, dlist, dlo, dhi, blk, ostage,
             sema, semb, semc, semd, semsa, semsb):
    wid = lax.axis_index("s") * _NC + lax.axis_index("c")
    tclo = wid * _TCPW
    lane = lax.iota(jnp.int32, _L)
    zeros16 = jnp.zeros((_L,), jnp.int32)
    tclov = _splat(tclo)
    tchiv = tclov + _TCPW

    pltpu.sync_copy(y_hbm.at[pl.ds(0, BATCH)], yva)

    sems = [sema, semb, semc, semd]
    ssems = [semsa, semsb]
    trash = _splat(BATCH) + lane

    def issue(s, pb, sem):
        tcs = dlist[pl.ds(s, _L)][0]
        off = pl.multiple_of(tcs * 128, 128)
        pltpu.async_copy(tT_hbm.at[:, pl.ds(off, 128)],
                         blk.at[pb, :, pl.ds(0, 128)], sem)

    def wait(s, pb, sem):
        tcs = dlist[pl.ds(s, _L)][0]
        off = pl.multiple_of(tcs * 128, 128)
        pltpu.make_async_copy(tT_hbm.at[:, pl.ds(off, 128)],
                              blk.at[pb, :, pl.ds(0, 128)], sem).wait()

    def sc_issue(chunk, sl):
        pltpu.async_copy(ostage.at[sl],
                         out_hbm.at[bperm.at[chunk]], ssems[sl])

    def sc_wait(sl):
        pltpu.make_async_copy(ostage.at[sl],
                              out_hbm.at[bperm.at[0]], ssems[sl]).wait()

    def do_round(rnd, mtot_in, windowed):
        rlo = rnd * _CAP
        rlov = _splat(rlo)
        rhiv = rlov + _CAP

        def zero_cnt(i, c):
            cnt[pl.ds(i * _L, _L)] = zeros16
            return c

        lax.fori_loop(0, _NBK // _L, zero_cnt, jnp.int32(0))

        def fill_bperm(i, c):
            bperm[i % 16, pl.ds((i // 16) * _L, _L)] = trash
            return c

        lax.fori_loop(0, _CAP // _L, fill_bperm, jnp.int32(0))

        def scan_body(i, runv):
            yv = yva[pl.ds(i * _L, _L)]
            tcv = lax.shift_right_logical(yv, 7)
            mk = (tcv >= tclov) & (tcv < tchiv)
            if windowed:
                mki = jnp.where(mk, jnp.int32(1), jnp.int32(0))
                gpos = runv + plsc.cumsum(mki) - jnp.int32(1)
                wmk = mk & (gpos >= rlov) & (gpos < rhiv)
                loff = jnp.clip(runv[0] - rlo, 0, _CAP)
            else:
                wmk = mk
                loff = jnp.minimum(runv[0], jnp.int32(_CAP))
            plsc.store_compressed(ylist.at[pl.ds(loff, _L)], yv, mask=wmk)
            plsc.store_compressed(blist.at[pl.ds(loff, _L)], i * _L + lane,
                                  mask=wmk)
            return runv + plsc.all_reduce_population_count(mk)

        mtot = lax.fori_loop(0, BATCH // _L, scan_body, zeros16)[0]
        m = jnp.minimum(jnp.int32(_CAP),
                        jnp.maximum(mtot, mtot_in) - rlo)

        nev = lax.shift_right_logical(m + jnp.int32(_L - 1), 4)
        mv = _splat(m)

        def count_body(v, c):
            valid = (v * _L + lane) < mv
            yvv = ylist[pl.ds(v * _L, _L)]
            bk = lax.shift_right_logical(yvv, 7) - tclov
            cnts, lastm = plsc.scan_count(bk, mask=valid)
            w = plsc.load_gather(cnt, [bk], mask=lastm)
            plsc.store_scatter(cnt, [bk], w + cnts, mask=lastm)
            return c

        lax.fori_loop(0, nev, count_body, jnp.int32(0))

        def boff_body(i, runv):
            v = cnt[pl.ds(i * _L, _L)]
            cs = plsc.cumsum(v)
            ex = runv + cs - v
            boffa[pl.ds(i * _L, _L)] = ex
            boffb[pl.ds(i * _L, _L)] = ex
            return _splat((runv + cs)[15])

        lax.fori_loop(0, _NBK // _L, boff_body, zeros16)

        def compact_body(i, runv):
            cv = cnt[pl.ds(i * _L, _L)]
            mk = cv > 0
            mki = jnp.where(mk, jnp.int32(1), jnp.int32(0))
            cs = plsc.cumsum(mki)
            slots = runv + cs - mki
            tcg = tclov + i * _L + lane
            bo = boffa[pl.ds(i * _L, _L)]
            plsc.store_scatter(dlist, [slots], tcg, mask=mk)
            plsc.store_scatter(dlo, [slots], bo, mask=mk)
            plsc.store_scatter(dhi, [slots], bo + cv, mask=mk)
            return _splat((runv + cs)[15])

        nblocks = lax.fori_loop(0, _NBK // _L, compact_body, zeros16)[0]

        def place_body(v, c):
            valid = (v * _L + lane) < mv
            yvv = ylist[pl.ds(v * _L, _L)]
            bk = lax.shift_right_logical(yvv, 7) - tclov
            cnts, lastm = plsc.scan_count(bk, mask=valid)
            base = plsc.load_gather(boffb, [bk], mask=valid)
            pos = base + cnts - jnp.int32(1)
            plsc.store_scatter(boffb, [bk], base + cnts, mask=lastm)
            plsc.store_scatter(perm, [pos], yvv, mask=valid)
            bvv = blist[pl.ds(v * _L, _L)]
            plsc.store_scatter(
                bperm,
                [lax.shift_right_logical(pos, 7), pos & jnp.int32(127)],
                bvv, mask=valid)
            return c

        lax.fori_loop(0, nev, place_body, jnp.int32(0))

        def process(s, pb):
            lo_e = dlo[pl.ds(s, _L)][0]
            hi_e = dhi[pl.ds(s, _L)][0]
            pbv = _splat(pb)

            def ebody(pos, c):
                r = pos & jnp.int32(127)
                chunk = lax.shift_right_logical(pos, 7)
                sl = chunk & jnp.int32(1)

                @pl.when((r == 0) & (pos >= 256))
                def _():
                    @pl.when(sl == 0)
                    def _():
                        sc_wait(0)

                    @pl.when(sl == 1)
                    def _():
                        sc_wait(1)

                ye = perm[pl.ds(pos, _L)][0]
                colv = _splat(ye & jnp.int32(127))
                gs = []
                acc = jnp.zeros((_L,), jnp.float32)
                for g in range(4):
                    gv = plsc.load_gather(blk, [pbv, lane + _L * g, colv])
                    gs.append(gv)
                    acc = acc + gv * gv
                scale = _rsqrt16(_splat(jnp.sum(acc), jnp.float32))
                for g in range(4):
                    ostage[sl, r, pl.ds(_L * g, _L)] = gs[g] * scale

                @pl.when(r == 127)
                def _():
                    @pl.when(sl == 0)
                    def _():
                        sc_issue(chunk, 0)

                    @pl.when(sl == 1)
                    def _():
                        sc_issue(chunk, 1)

                return c

            lax.fori_loop(lo_e, hi_e, ebody, jnp.int32(0))

        for j in range(_RING):
            @pl.when(jnp.int32(j) < nblocks)
            def _(j=j):
                issue(jnp.int32(j), j, sems[j])

        nring = lax.div(nblocks + jnp.int32(_RING - 1), jnp.int32(_RING))

        def ring_body(q, c):
            s0 = q * _RING
            for j in range(_RING):
                sj = s0 + j

                @pl.when(sj < nblocks)
                def _(j=j, sj=sj):
                    wait(sj, j, sems[j])
                    process(sj, j)

                    @pl.when(sj + _RING < nblocks)
                    def _(j=j, sj=sj):
                        issue(sj + _RING, j, sems[j])

            return c

        lax.fori_loop(0, nring, ring_body, jnp.int32(0))

        lastc = lax.shift_right_logical(m, 7)
        lsl = lastc & jnp.int32(1)

        @pl.when((m & jnp.int32(127)) != 0)
        def _():
            @pl.when(lsl == 0)
            def _():
                sc_issue(lastc, 0)

            @pl.when(lsl == 1)
            def _():
                sc_issue(lastc, 1)

        nchunks = lax.shift_right_logical(m + jnp.int32(127), 7)
        for p in range(2):
            @pl.when((nchunks >= 1) & (((nchunks - 1) & jnp.int32(1)) == p))
            def _(p=p):
                sc_wait(p)

            @pl.when((nchunks >= 2) & (((nchunks - 2) & jnp.int32(1)) == p))
            def _(p=p):
                sc_wait(p)

        return mtot

    mtot0 = do_round(jnp.int32(0), jnp.int32(0), windowed=False)

    def more_cond(st):
        rnd, mtot = st
        return (rnd + 1) * _CAP < mtot

    def more_body(st):
        rnd, mtot = st
        rnd = rnd + 1
        do_round(rnd, mtot, windowed=True)
        return (rnd, mtot)

    lax.while_loop(more_cond, more_body, (jnp.int32(0), mtot0))


@jax.jit
def kernel(y, table):
    mesh = plsc.VectorSubcoreMesh(core_axis_name="c", subcore_axis_name="s")
    k = functools.partial(
        pl.kernel,
        mesh=mesh,
        compiler_params=pltpu.CompilerParams(needs_layout_passes=False),
        out_type=jax.ShapeDtypeStruct((_OPAD, 2 * EMBED_DIM), jnp.float32),
        scratch_types=[
            pltpu.VMEM((BATCH,), jnp.int32),
            pltpu.VMEM((_CAP + _L,), jnp.int32),
            pltpu.VMEM((_CAP + _L,), jnp.int32),
            pltpu.VMEM((_NBK,), jnp.int32),
            pltpu.VMEM((_NBK,), jnp.int32),
            pltpu.VMEM((_NBK,), jnp.int32),
            pltpu.VMEM((_CAP + _L,), jnp.int32),
            pltpu.VMEM((_CAP // 128, 128), jnp.int32),
            pltpu.VMEM((_NBK,), jnp.int32),
            pltpu.VMEM((_NBK,), jnp.int32),
            pltpu.VMEM((_NBK,), jnp.int32),
            pltpu.VMEM((_RING, EMBED_DIM, _BPAD), jnp.float32),
            pltpu.VMEM((2, 128, 2 * EMBED_DIM), jnp.float32),
            pltpu.SemaphoreType.DMA,
            pltpu.SemaphoreType.DMA,
            pltpu.SemaphoreType.DMA,
            pltpu.SemaphoreType.DMA,
            pltpu.SemaphoreType.DMA,
            pltpu.SemaphoreType.DMA,
        ],
    )(_sc_body)
    out2 = k(y.astype(jnp.int32), table.T)
    return out2[:BATCH, :EMBED_DIM]
